# baseline (speedup 1.0000x reference)
import functools

import jax
import jax.numpy as jnp
from jax import lax
from jax.experimental import pallas as pl
from jax.experimental.pallas import tpu as pltpu
from jax.experimental.pallas import tpu_sc as plsc

N = 10000
E = 320000
D = 128
NC = 2
NS = 16
NW = NC * NS
EPW = E // NW
CG = 80
ROUNDS = EPW // CG
NPAD = 10240
RPT = NPAD // NS
ZR = 16

_SELU_ALPHA = 1.6732632423543772
_SELU_SCALE = 1.0507009873554805


def _selu(x):
    return _SELU_SCALE * jnp.where(x > 0, x, _SELU_ALPHA * (jnp.exp(x) - 1.0))


def _bmm(x, w):
    return jnp.dot(x.astype(jnp.bfloat16), w.astype(jnp.bfloat16),
                   preferred_element_type=jnp.float32)


def _node_init_body(v_ref, c_ref, Wi_ref, bi_ref, Wc_ref, bc_ref,
                    Ws_ref, be1_ref, Wd_ref, h_ref, p_ref, q_ref):
    h = (jnp.dot(v_ref[...], Wi_ref[...], preferred_element_type=jnp.float32)
         + bi_ref[...]
         + jnp.dot(c_ref[...], Wc_ref[...], preferred_element_type=jnp.float32)
         + bc_ref[...])
    h_ref[...] = h
    p_ref[...] = jnp.dot(h, Ws_ref[...], preferred_element_type=jnp.float32) + be1_ref[...]
    q_ref[...] = jnp.dot(h, Wd_ref[...], preferred_element_type=jnp.float32)


def _node_init(v, c, Wi, bi, Wc, bc, Ws, be1, Wd):
    BN = 2000
    grid = (N // BN,)
    row = pl.BlockSpec((BN, D), lambda i: (i, 0))
    full = pl.BlockSpec((D, D), lambda i: (0, 0))
    bias = pl.BlockSpec((1, D), lambda i: (0, 0))
    return pl.pallas_call(
        _node_init_body,
        grid=grid,
        in_specs=[row, row, full, bias, full, bias, full, bias, full],
        out_specs=[row, row, row],
        out_shape=[jax.ShapeDtypeStruct((N, D), jnp.float32)] * 3,
    )(v, c, Wi, bi, Wc, bc, Ws, be1, Wd)


def _edge_body_first(el_ref, g_ref, Wed_ref, bed_ref, We_ref, We2_ref,
                     be2_ref, out_ref):
    e = _bmm(el_ref[...], Wed_ref[...]) + bed_ref[...]
    t = _selu(_bmm(e, We_ref[...]) + g_ref[...])
    out_ref[...] = _bmm(t, We2_ref[...]) + be2_ref[...] + e


def _edge_body_mid(e_ref, g_ref, We_ref, We2_ref, be2_ref, out_ref):
    e = e_ref[...]
    t = _selu(_bmm(e, We_ref[...]) + g_ref[...])
    out_ref[...] = _bmm(t, We2_ref[...]) + be2_ref[...] + e


def _edge_mlp(e_in, G, We, We2, be2, Wed=None, bed=None):
    BE = 2560
    grid = (E // BE,)
    row = pl.BlockSpec((BE, D), lambda i: (i, 0))
    full = pl.BlockSpec((D, D), lambda i: (0, 0))
    bias = pl.BlockSpec((1, D), lambda i: (0, 0))
    if Wed is not None:
        return pl.pallas_call(
            _edge_body_first,
            grid=grid,
            in_specs=[row, row, full, bias, full, full, bias],
            out_specs=row,
            out_shape=jax.ShapeDtypeStruct((E, D), jnp.float32),
        )(e_in, G, Wed, bed, We, We2, be2)
    return pl.pallas_call(
        _edge_body_mid,
        grid=grid,
        in_specs=[row, row, full, full, bias],
        out_specs=row,
        out_shape=jax.ShapeDtypeStruct((E, D), jnp.float32),
    )(e_in, G, We, We2, be2)


def _node_mid_body(h_ref, a0_ref, a1_ref, c0_ref, c1_ref, Wh_ref, Wa_ref,
                   bn1_ref, Wn2_ref, bn2_ref, Ws_ref, be1_ref, Wd_ref,
                   h_out, p_out, q_out):
    cnt = c0_ref[...][:, 0:1] + c1_ref[...][:, 0:1]
    agg = (a0_ref[...] + a1_ref[...]) / jnp.maximum(cnt, 1.0)
    h = h_ref[...]
    t = _selu(jnp.dot(h, Wh_ref[...], preferred_element_type=jnp.float32)
              + jnp.dot(agg, Wa_ref[...], preferred_element_type=jnp.float32)
              + bn1_ref[...])
    hn = jnp.dot(t, Wn2_ref[...], preferred_element_type=jnp.float32) + bn2_ref[...] + h
    h_out[...] = hn
    p_out[...] = jnp.dot(hn, Ws_ref[...], preferred_element_type=jnp.float32) + be1_ref[...]
    q_out[...] = jnp.dot(hn, Wd_ref[...], preferred_element_type=jnp.float32)


def _node_mid(h, a0, a1, c0, c1, Wh, Wa, bn1, Wn2, bn2, Ws, be1, Wd):
    BN = 2000
    grid = (N // BN,)
    row = pl.BlockSpec((BN, D), lambda i: (i, 0))
    crow = pl.BlockSpec((BN, D), lambda i: (i, 0))
    full = pl.BlockSpec((D, D), lambda i: (0, 0))
    bias = pl.BlockSpec((1, D), lambda i: (0, 0))
    return pl.pallas_call(
        _node_mid_body,
        grid=grid,
        in_specs=[row, row, row, crow, crow, full, full, bias, full, bias,
                  full, bias, full],
        out_specs=[row, row, row],
        out_shape=[jax.ShapeDtypeStruct((N, D), jnp.float32)] * 3,
    )(h, a0, a1, c0, c1, Wh, Wa, bn1, Wn2, bn2, Ws, be1, Wd)


def _node_final_body(h_ref, a0_ref, a1_ref, c0_ref, c1_ref, Wh_ref, Wa_ref,
                     bn1_ref, Wn2_ref, bn2_ref, Wo_ref, bo_ref, out_ref):
    cnt = c0_ref[...][:, 0:1] + c1_ref[...][:, 0:1]
    agg = (a0_ref[...] + a1_ref[...]) / jnp.maximum(cnt, 1.0)
    h = h_ref[...]
    t = _selu(jnp.dot(h, Wh_ref[...], preferred_element_type=jnp.float32)
              + jnp.dot(agg, Wa_ref[...], preferred_element_type=jnp.float32)
              + bn1_ref[...])
    hn = jnp.dot(t, Wn2_ref[...], preferred_element_type=jnp.float32) + bn2_ref[...] + h
    out_ref[...] = jnp.dot(hn, Wo_ref[...], preferred_element_type=jnp.float32) + bo_ref[...]


def _node_final(h, a0, a1, c0, c1, Wh, Wa, bn1, Wn2, bn2, Wo, bo):
    BN = 2000
    grid = (N // BN,)
    row = pl.BlockSpec((BN, D), lambda i: (i, 0))
    crow = pl.BlockSpec((BN, D), lambda i: (i, 0))
    full = pl.BlockSpec((D, D), lambda i: (0, 0))
    bias = pl.BlockSpec((1, D), lambda i: (0, 0))
    return pl.pallas_call(
        _node_final_body,
        grid=grid,
        in_specs=[row, row, row, crow, crow, full, full, bias, full, bias,
                  full, bias],
        out_specs=row,
        out_shape=jax.ShapeDtypeStruct((N, D), jnp.float32),
    )(h, a0, a1, c0, c1, Wh, Wa, bn1, Wn2, bn2, Wo, bo)


@functools.cache
def _get_sc_gather():
    mesh = plsc.VectorSubcoreMesh(core_axis_name="c", subcore_axis_name="s")

    @functools.partial(
        pl.kernel,
        out_type=jax.ShapeDtypeStruct((E, D), jnp.float32),
        mesh=mesh,
        scratch_types=[
            pltpu.VMEM((ROUNDS, CG), jnp.int32),
            pltpu.VMEM((ROUNDS, CG), jnp.int32),
            pltpu.VMEM((2, CG, D), jnp.float32),
            pltpu.VMEM((2, CG, D), jnp.float32),
            pltpu.VMEM((2, CG, D), jnp.float32),
            pltpu.SemaphoreType.DMA,
            pltpu.SemaphoreType.DMA,
            pltpu.SemaphoreType.DMA,
            pltpu.SemaphoreType.DMA,
            pltpu.SemaphoreType.DMA,
            pltpu.SemaphoreType.DMA,
        ],
    )
    def _sc_gather(p_hbm, q_hbm, src2_hbm, dst2_hbm, g_hbm, idxs, idxd,
                   bufp, bufq, bufs, semp0, semp1, semq0, semq1, sems0,
                   sems1):
        wid = lax.axis_index("s") * NC + lax.axis_index("c")
        base0 = wid * EPW
        semp = (semp0, semp1)
        semq = (semq0, semq1)
        sems = (sems0, sems1)

        pltpu.sync_copy(src2_hbm.at[wid], idxs)
        pltpu.sync_copy(dst2_hbm.at[wid], idxd)

        def fire(b, r):
            cpp = pltpu.async_copy(p_hbm.at[idxs.at[r]], bufp.at[b], semp[b])
            cpq = pltpu.async_copy(q_hbm.at[idxd.at[r]], bufq.at[b], semq[b])
            return cpp, cpq

        def wait_gather(b, r):
            pltpu.make_async_copy(p_hbm.at[idxs.at[r]], bufp.at[b], semp[b]).wait()
            pltpu.make_async_copy(q_hbm.at[idxd.at[r]], bufq.at[b], semq[b]).wait()

        def wait_store(b, r):
            base = pl.multiple_of(base0 + r * CG, CG)
            pltpu.make_async_copy(bufs.at[b], g_hbm.at[pl.ds(base, CG)],
                                  sems[b]).wait()

        def fire_store(b, r):
            base = pl.multiple_of(base0 + r * CG, CG)
            return pltpu.async_copy(bufs.at[b], g_hbm.at[pl.ds(base, CG)],
                                    sems[b])

        def add(b):
            def add_row(i, carry):
                for j in range(D // 16):
                    sl = pl.ds(j * 16, 16)
                    bufs[b, i, sl] = bufp[b, i, sl] + bufq[b, i, sl]
                return carry

            lax.fori_loop(0, CG, add_row, 0)

        fire(0, 0)
        fire(1, 1)

        def step(g, carry):
            for b in range(2):
                r = g + b
                wait_gather(b, r)

                @pl.when(r >= 2)
                def _():
                    wait_store(b, r - 2)

                add(b)
                fire_store(b, r)

                @pl.when(r + 2 < ROUNDS)
                def _():
                    fire(b, r + 2)
            return carry

        lax.fori_loop(0, (ROUNDS - 1) // 2, lambda g, c: step(g * 2, c), 0)
        r = ROUNDS - 1
        wait_gather(0, r)
        wait_store(0, r - 2)
        add(0)
        fire_store(0, r)
        wait_store(1, r - 1)
        wait_store(0, r)

    return _sc_gather


@functools.cache
def _get_sc_scatter():
    out_type = (
        jax.ShapeDtypeStruct((NPAD, D), jnp.float32),
        jax.ShapeDtypeStruct((NPAD, D), jnp.float32),
    )
    scratch = [
        pltpu.VMEM((2, CG), jnp.int32),
        pltpu.VMEM((2, CG, D), jnp.float32),
        pltpu.VMEM((ZR, D), jnp.float32),
        pltpu.VMEM_SHARED((NPAD, D), jnp.float32),
    ] + [pltpu.SemaphoreType.DMA] * 4
    mesh = plsc.VectorSubcoreMesh(core_axis_name="c", subcore_axis_name="s")

    @functools.partial(pl.kernel, out_type=out_type, mesh=mesh,
                       scratch_types=scratch)
    def _scatter(e_hbm, dst2_hbm, agg0, agg1, idxr, ebuf, zbuf, acc,
                 l0, l1, i0, i1):
        semld = (l0, l1)
        semix = (i0, i1)
        cid = lax.axis_index("c")
        sid = lax.axis_index("s")
        wid = sid * NC + cid
        base0 = wid * EPW
        row0 = sid * RPT

        def zrow(i, carry):
            for j in range(D // 16):
                zbuf[i, pl.ds(j * 16, 16)] = jnp.zeros((16,), jnp.float32)
            return carry

        lax.fori_loop(0, ZR, zrow, 0)
        for k in range(RPT // ZR):
            pltpu.sync_copy(zbuf, acc.at[pl.ds(row0 + k * ZR, ZR)])

        plsc.subcore_barrier()

        def fire_load(b, r):
            base = pl.multiple_of(base0 + r * CG, CG)
            pltpu.async_copy(dst2_hbm.at[wid].at[pl.ds(r, 1)],
                             idxr.at[pl.ds(b, 1)], semix[b])
            return pltpu.async_copy(e_hbm.at[pl.ds(base, CG)], ebuf.at[b],
                                    semld[b])

        def wait_load(b, r):
            base = pl.multiple_of(base0 + r * CG, CG)
            pltpu.make_async_copy(e_hbm.at[pl.ds(base, CG)], ebuf.at[b],
                                  semld[b]).wait()
            pltpu.make_async_copy(dst2_hbm.at[wid].at[pl.ds(r, 1)],
                                  idxr.at[pl.ds(b, 1)], semix[b]).wait()

        fire_load(0, 0)
        fire_load(1, 1)

        def step(g, carry):
            for b in range(2):
                r = g * 2 + b
                wait_load(b, r)
                pltpu.sync_copy(ebuf.at[b], acc.at[idxr.at[b]], add=True)

                @pl.when(r + 2 < ROUNDS)
                def _():
                    fire_load(b, r + 2)
            return carry

        lax.fori_loop(0, (ROUNDS - 1) // 2, step, 0)
        wait_load(0, ROUNDS - 1)
        pltpu.sync_copy(ebuf.at[0], acc.at[idxr.at[0]], add=True)
        plsc.subcore_barrier()

        for k in range(RPT // ZR):
            rs = pl.ds(row0 + k * ZR, ZR)
            pltpu.sync_copy(acc.at[rs], zbuf)

            @pl.when(cid == 0)
            def _():
                pltpu.sync_copy(zbuf, agg0.at[rs])

            @pl.when(cid == 1)
            def _():
                pltpu.sync_copy(zbuf, agg1.at[rs])

    return _scatter


@functools.cache
def _get_sc_counts():
    out_type = (
        jax.ShapeDtypeStruct((NPAD, D), jnp.float32),
        jax.ShapeDtypeStruct((NPAD, D), jnp.float32),
    )
    scratch = [
        pltpu.VMEM((2, CG), jnp.int32),
        pltpu.VMEM((CG, D), jnp.float32),
        pltpu.VMEM((ZR, D), jnp.float32),
        pltpu.VMEM_SHARED((NPAD, D), jnp.float32),
    ] + [pltpu.SemaphoreType.DMA] * 2
    mesh = plsc.VectorSubcoreMesh(core_axis_name="c", subcore_axis_name="s")

    @functools.partial(pl.kernel, out_type=out_type, mesh=mesh,
                       scratch_types=scratch)
    def _counts(dst2_hbm, cnt0, cnt1, idxr, ones, zbuf, acc, i0, i1):
        semix = (i0, i1)
        cid = lax.axis_index("c")
        sid = lax.axis_index("s")
        wid = sid * NC + cid
        row0 = sid * RPT

        def zrow(i, carry):
            for j in range(D // 16):
                zbuf[i, pl.ds(j * 16, 16)] = jnp.zeros((16,), jnp.float32)
            return carry

        lax.fori_loop(0, ZR, zrow, 0)

        def onesrow(i, carry):
            for j in range(D // 16):
                ones[i, pl.ds(j * 16, 16)] = jnp.ones((16,), jnp.float32)
            return carry

        lax.fori_loop(0, CG, onesrow, 0)
        for k in range(RPT // ZR):
            pltpu.sync_copy(zbuf, acc.at[pl.ds(row0 + k * ZR, ZR)])
        plsc.subcore_barrier()

        def fire_idx(b, r):
            return pltpu.async_copy(dst2_hbm.at[wid].at[pl.ds(r, 1)],
                                    idxr.at[pl.ds(b, 1)], semix[b])

        def wait_idx(b, r):
            pltpu.make_async_copy(dst2_hbm.at[wid].at[pl.ds(r, 1)],
                                  idxr.at[pl.ds(b, 1)], semix[b]).wait()

        fire_idx(0, 0)
        fire_idx(1, 1)

        def step(g, carry):
            for b in range(2):
                r = g * 2 + b
                wait_idx(b, r)
                pltpu.sync_copy(ones, acc.at[idxr.at[b]], add=True)

                @pl.when(r + 2 < ROUNDS)
                def _():
                    fire_idx(b, r + 2)
            return carry

        lax.fori_loop(0, (ROUNDS - 1) // 2, step, 0)
        wait_idx(0, ROUNDS - 1)
        pltpu.sync_copy(ones, acc.at[idxr.at[0]], add=True)
        plsc.subcore_barrier()

        for k in range(RPT // ZR):
            rs = pl.ds(row0 + k * ZR, ZR)
            pltpu.sync_copy(acc.at[rs], zbuf)

            @pl.when(cid == 0)
            def _():
                pltpu.sync_copy(zbuf, cnt0.at[rs])

            @pl.when(cid == 1)
            def _():
                pltpu.sync_copy(zbuf, cnt1.at[rs])

    return _counts


def kernel(v, c_latent, e_latent, params, edge_index, batch):
    src = edge_index[0].astype(jnp.int32)
    dst = edge_index[1].astype(jnp.int32)
    p = params
    blk0, blk1 = p['block0'], p['block1']

    def b2(b):
        return b.reshape(1, D)

    We1_0, We1_1 = blk0['We1'], blk1['We1']
    Wn1_0, Wn1_1 = blk0['Wn1'], blk1['Wn1']
    Wo = jnp.pad(p['W_out'], ((0, 0), (0, D - p['W_out'].shape[1])))
    bo = jnp.pad(p['b_out'], (0, D - p['b_out'].shape[0])).reshape(1, D)

    h0, P0, Q0 = _node_init(v, c_latent, p['W_in'], b2(p['b_in']),
                            p['W_cond'], b2(p['b_cond']),
                            We1_0[D:2 * D], b2(blk0['be1']), We1_0[2 * D:])
    src2 = src.reshape(NW, ROUNDS, CG)
    dst2 = dst.reshape(NW, ROUNDS, CG)
    G0 = _get_sc_gather()(P0, Q0, src2, dst2)
    e1 = _edge_mlp(e_latent, G0, We1_0[:D], blk0['We2'], b2(blk0['be2']),
                   Wed=p['W_edge'], bed=b2(p['b_edge']))
    c0, c1 = _get_sc_counts()(dst2)
    a0, a1 = _get_sc_scatter()(e1, dst2)
    h1, P1, Q1 = _node_mid(h0, a0, a1, c0, c1, Wn1_0[:D], Wn1_0[D:],
                           b2(blk0['bn1']), blk0['Wn2'], b2(blk0['bn2']),
                           We1_1[D:2 * D], b2(blk1['be1']), We1_1[2 * D:])
    G1 = _get_sc_gather()(P1, Q1, src2, dst2)
    e2 = _edge_mlp(e1, G1, We1_1[:D], blk1['We2'], b2(blk1['be2']))
    a0b, a1b = _get_sc_scatter()(e2, dst2)
    out_pad = _node_final(h1, a0b, a1b, c0, c1, Wn1_1[:D], Wn1_1[D:],
                          b2(blk1['bn1']), blk1['Wn2'], b2(blk1['bn2']),
                          Wo, bo)
    return out_pad[:, :3]


# speedup vs baseline: 1.0345x; 1.0345x over previous
import functools

import jax
import jax.numpy as jnp
from jax import lax
from jax.experimental import pallas as pl
from jax.experimental.pallas import tpu as pltpu
from jax.experimental.pallas import tpu_sc as plsc

N = 10000
E = 320000
D = 128
NC = 2
NS = 16
NW = NC * NS
EPW = E // NW
CG = 80
ROUNDS = EPW // CG
NPAD = 10240
RPT = NPAD // NS
ZR = 16

_SELU_ALPHA = 1.6732632423543772
_SELU_SCALE = 1.0507009873554805


def _selu(x):
    return _SELU_SCALE * jnp.where(x > 0, x, _SELU_ALPHA * (jnp.exp(x) - 1.0))


def _bmm(x, w):
    return jnp.dot(x.astype(jnp.bfloat16), w.astype(jnp.bfloat16),
                   preferred_element_type=jnp.float32)


def _node_init_body(v_ref, c_ref, Wi_ref, bi_ref, Wc_ref, bc_ref,
                    Ws_ref, be1_ref, Wd_ref, h_ref, p_ref, q_ref):
    h = (jnp.dot(v_ref[...], Wi_ref[...], preferred_element_type=jnp.float32)
         + bi_ref[...]
         + jnp.dot(c_ref[...], Wc_ref[...], preferred_element_type=jnp.float32)
         + bc_ref[...])
    h_ref[...] = h
    p_ref[...] = jnp.dot(h, Ws_ref[...], preferred_element_type=jnp.float32) + be1_ref[...]
    q_ref[...] = jnp.dot(h, Wd_ref[...], preferred_element_type=jnp.float32)


def _node_init(v, c, Wi, bi, Wc, bc, Ws, be1, Wd):
    BN = 2000
    grid = (N // BN,)
    row = pl.BlockSpec((BN, D), lambda i: (i, 0))
    full = pl.BlockSpec((D, D), lambda i: (0, 0))
    bias = pl.BlockSpec((1, D), lambda i: (0, 0))
    return pl.pallas_call(
        _node_init_body,
        grid=grid,
        in_specs=[row, row, full, bias, full, bias, full, bias, full],
        out_specs=[row, row, row],
        out_shape=[jax.ShapeDtypeStruct((N, D), jnp.float32)] * 3,
    )(v, c, Wi, bi, Wc, bc, Ws, be1, Wd)


def _edge_body_first(el_ref, g_ref, Wed_ref, bed_ref, We_ref, We2_ref,
                     be2_ref, out_ref):
    e = _bmm(el_ref[...], Wed_ref[...]) + bed_ref[...]
    t = _selu(_bmm(e, We_ref[...]) + g_ref[...])
    out_ref[...] = _bmm(t, We2_ref[...]) + be2_ref[...] + e


def _edge_body_mid(e_ref, g_ref, We_ref, We2_ref, be2_ref, out_ref):
    e = e_ref[...]
    t = _selu(_bmm(e, We_ref[...]) + g_ref[...])
    out_ref[...] = _bmm(t, We2_ref[...]) + be2_ref[...] + e


def _edge_mlp(e_in, G, We, We2, be2, Wed=None, bed=None):
    BE = 2560
    grid = (E // BE,)
    row = pl.BlockSpec((BE, D), lambda i: (i, 0))
    full = pl.BlockSpec((D, D), lambda i: (0, 0))
    bias = pl.BlockSpec((1, D), lambda i: (0, 0))
    if Wed is not None:
        return pl.pallas_call(
            _edge_body_first,
            grid=grid,
            in_specs=[row, row, full, bias, full, full, bias],
            out_specs=row,
            out_shape=jax.ShapeDtypeStruct((E, D), jnp.float32),
        )(e_in, G, Wed, bed, We, We2, be2)
    return pl.pallas_call(
        _edge_body_mid,
        grid=grid,
        in_specs=[row, row, full, full, bias],
        out_specs=row,
        out_shape=jax.ShapeDtypeStruct((E, D), jnp.float32),
    )(e_in, G, We, We2, be2)


def _node_mid_body(h_ref, a0_ref, a1_ref, c0_ref, c1_ref, Wh_ref, Wa_ref,
                   bn1_ref, Wn2_ref, bn2_ref, Ws_ref, be1_ref, Wd_ref,
                   h_out, p_out, q_out):
    cnt = c0_ref[...][:, 0:1] + c1_ref[...][:, 0:1]
    agg = (a0_ref[...] + a1_ref[...]) / jnp.maximum(cnt, 1.0)
    h = h_ref[...]
    t = _selu(jnp.dot(h, Wh_ref[...], preferred_element_type=jnp.float32)
              + jnp.dot(agg, Wa_ref[...], preferred_element_type=jnp.float32)
              + bn1_ref[...])
    hn = jnp.dot(t, Wn2_ref[...], preferred_element_type=jnp.float32) + bn2_ref[...] + h
    h_out[...] = hn
    p_out[...] = jnp.dot(hn, Ws_ref[...], preferred_element_type=jnp.float32) + be1_ref[...]
    q_out[...] = jnp.dot(hn, Wd_ref[...], preferred_element_type=jnp.float32)


def _node_mid(h, a0, a1, c0, c1, Wh, Wa, bn1, Wn2, bn2, Ws, be1, Wd):
    BN = 2000
    grid = (N // BN,)
    row = pl.BlockSpec((BN, D), lambda i: (i, 0))
    crow = pl.BlockSpec((BN, D), lambda i: (i, 0))
    full = pl.BlockSpec((D, D), lambda i: (0, 0))
    bias = pl.BlockSpec((1, D), lambda i: (0, 0))
    return pl.pallas_call(
        _node_mid_body,
        grid=grid,
        in_specs=[row, row, row, crow, crow, full, full, bias, full, bias,
                  full, bias, full],
        out_specs=[row, row, row],
        out_shape=[jax.ShapeDtypeStruct((N, D), jnp.float32)] * 3,
    )(h, a0, a1, c0, c1, Wh, Wa, bn1, Wn2, bn2, Ws, be1, Wd)


def _node_final_body(h_ref, a0_ref, a1_ref, c0_ref, c1_ref, Wh_ref, Wa_ref,
                     bn1_ref, Wn2_ref, bn2_ref, Wo_ref, bo_ref, out_ref):
    cnt = c0_ref[...][:, 0:1] + c1_ref[...][:, 0:1]
    agg = (a0_ref[...] + a1_ref[...]) / jnp.maximum(cnt, 1.0)
    h = h_ref[...]
    t = _selu(jnp.dot(h, Wh_ref[...], preferred_element_type=jnp.float32)
              + jnp.dot(agg, Wa_ref[...], preferred_element_type=jnp.float32)
              + bn1_ref[...])
    hn = jnp.dot(t, Wn2_ref[...], preferred_element_type=jnp.float32) + bn2_ref[...] + h
    out_ref[...] = jnp.dot(hn, Wo_ref[...], preferred_element_type=jnp.float32) + bo_ref[...]


def _node_final(h, a0, a1, c0, c1, Wh, Wa, bn1, Wn2, bn2, Wo, bo):
    BN = 2000
    grid = (N // BN,)
    row = pl.BlockSpec((BN, D), lambda i: (i, 0))
    crow = pl.BlockSpec((BN, D), lambda i: (i, 0))
    full = pl.BlockSpec((D, D), lambda i: (0, 0))
    bias = pl.BlockSpec((1, D), lambda i: (0, 0))
    return pl.pallas_call(
        _node_final_body,
        grid=grid,
        in_specs=[row, row, row, crow, crow, full, full, bias, full, bias,
                  full, bias],
        out_specs=row,
        out_shape=jax.ShapeDtypeStruct((N, D), jnp.float32),
    )(h, a0, a1, c0, c1, Wh, Wa, bn1, Wn2, bn2, Wo, bo)


@functools.cache
def _get_sc_gather():
    mesh = plsc.VectorSubcoreMesh(core_axis_name="c", subcore_axis_name="s")

    @functools.partial(
        pl.kernel,
        out_type=jax.ShapeDtypeStruct((E, D), jnp.float32),
        mesh=mesh,
        scratch_types=[
            pltpu.VMEM((ROUNDS, CG), jnp.int32),
            pltpu.VMEM((ROUNDS, CG), jnp.int32),
            pltpu.VMEM((2, CG, D), jnp.float32),
            pltpu.VMEM((2, CG, D), jnp.float32),
            pltpu.VMEM((2, CG, D), jnp.float32),
            pltpu.SemaphoreType.DMA,
            pltpu.SemaphoreType.DMA,
            pltpu.SemaphoreType.DMA,
            pltpu.SemaphoreType.DMA,
            pltpu.SemaphoreType.DMA,
            pltpu.SemaphoreType.DMA,
        ],
    )
    def _sc_gather(p_hbm, q_hbm, src2_hbm, dst2_hbm, g_hbm, idxs, idxd,
                   bufp, bufq, bufs, semp0, semp1, semq0, semq1, sems0,
                   sems1):
        wid = lax.axis_index("s") * NC + lax.axis_index("c")
        base0 = wid * EPW
        semp = (semp0, semp1)
        semq = (semq0, semq1)
        sems = (sems0, sems1)

        pltpu.sync_copy(src2_hbm.at[wid], idxs)
        pltpu.sync_copy(dst2_hbm.at[wid], idxd)

        def fire(b, r):
            pltpu.async_copy(p_hbm.at[idxs.at[r]], bufp.at[b], semp[b])
            pltpu.async_copy(q_hbm.at[idxd.at[r]], bufq.at[b], semq[b])

        def wait_gather(b, r):
            pltpu.make_async_copy(p_hbm.at[idxs.at[r]], bufp.at[b], semp[b]).wait()
            pltpu.make_async_copy(q_hbm.at[idxd.at[r]], bufq.at[b], semq[b]).wait()

        def wait_store(b, r):
            base = pl.multiple_of(base0 + r * CG, CG)
            pltpu.make_async_copy(bufs.at[b], g_hbm.at[pl.ds(base, CG)],
                                  sems[b]).wait()

        def fire_store(b, r):
            base = pl.multiple_of(base0 + r * CG, CG)
            return pltpu.async_copy(bufs.at[b], g_hbm.at[pl.ds(base, CG)],
                                    sems[b])

        def add(b):
            def add_row(i, carry):
                for j in range(D // 16):
                    sl = pl.ds(j * 16, 16)
                    bufs[b, i, sl] = bufp[b, i, sl] + bufq[b, i, sl]
                return carry

            lax.fori_loop(0, CG, add_row, 0)

        fire(0, 0)
        fire(1, 1)

        def step(g, carry):
            for b in range(2):
                r = g * 2 + b
                wait_gather(b, r)

                @pl.when(r >= 2)
                def _():
                    wait_store(b, r - 2)

                add(b)
                fire_store(b, r)

                @pl.when(r + 2 < ROUNDS)
                def _():
                    fire(b, r + 2)
            return carry

        lax.fori_loop(0, (ROUNDS - 1) // 2, step, 0)
        r = ROUNDS - 1
        wait_gather(0, r)
        wait_store(0, r - 2)
        add(0)
        fire_store(0, r)
        wait_store(1, r - 1)
        wait_store(0, r)

    return _sc_gather


@functools.cache
def _get_sc_scatter():
    out_type = (
        jax.ShapeDtypeStruct((NPAD, D), jnp.float32),
        jax.ShapeDtypeStruct((NPAD, D), jnp.float32),
    )
    scratch = [
        pltpu.VMEM((3, CG), jnp.int32),
        pltpu.VMEM((3, CG, D), jnp.float32),
        pltpu.VMEM((ZR, D), jnp.float32),
        pltpu.VMEM_SHARED((NPAD, D), jnp.float32),
    ] + [pltpu.SemaphoreType.DMA] * 9
    mesh = plsc.VectorSubcoreMesh(core_axis_name="c", subcore_axis_name="s")

    @functools.partial(pl.kernel, out_type=out_type, mesh=mesh,
                       scratch_types=scratch)
    def _scatter(e_hbm, dst2_hbm, agg0, agg1, idxr, ebuf, zbuf, acc,
                 l0, l1, l2, i0, i1, i2, s0, s1, s2):
        semld = (l0, l1, l2)
        semix = (i0, i1, i2)
        semsc = (s0, s1, s2)
        cid = lax.axis_index("c")
        sid = lax.axis_index("s")
        wid = sid * NC + cid
        base0 = wid * EPW
        row0 = sid * RPT

        def zrow(i, carry):
            for j in range(D // 16):
                zbuf[i, pl.ds(j * 16, 16)] = jnp.zeros((16,), jnp.float32)
            return carry

        lax.fori_loop(0, ZR, zrow, 0)
        for k in range(RPT // ZR):
            pltpu.sync_copy(zbuf, acc.at[pl.ds(row0 + k * ZR, ZR)])

        plsc.subcore_barrier()

        def fire_load(b, r):
            base = pl.multiple_of(base0 + r * CG, CG)
            pltpu.async_copy(dst2_hbm.at[wid].at[pl.ds(r, 1)],
                             idxr.at[pl.ds(b, 1)], semix[b])
            return pltpu.async_copy(e_hbm.at[pl.ds(base, CG)], ebuf.at[b],
                                    semld[b])

        def wait_load(b, r):
            base = pl.multiple_of(base0 + r * CG, CG)
            pltpu.make_async_copy(e_hbm.at[pl.ds(base, CG)], ebuf.at[b],
                                  semld[b]).wait()
            pltpu.make_async_copy(dst2_hbm.at[wid].at[pl.ds(r, 1)],
                                  idxr.at[pl.ds(b, 1)], semix[b]).wait()

        def fire_scatter(b):
            pltpu.async_copy(ebuf.at[b], acc.at[idxr.at[b]], semsc[b],
                             add=True)

        def wait_scatter(b):
            pltpu.make_async_copy(ebuf.at[b], acc.at[idxr.at[b]],
                                  semsc[b]).wait()

        fire_load(0, 0)
        fire_load(1, 1)

        def visit(s, r, first, last):
            wait_load(s, r)
            fire_scatter(s)
            s1 = (s + 2) % 3
            if first:
                fire_load(s1, r + 2)
            else:
                wait_scatter(s1)

                @pl.when(r + 2 < ROUNDS)
                def _():
                    fire_load(s1, r + 2)
            if last:
                wait_scatter(s)

        def step(g, carry):
            for b in range(3):
                visit(b, g * 3 + b, False, False)
            return carry

        visit(0, 0, True, False)
        visit(1, 1, False, False)
        visit(2, 2, False, False)
        lax.fori_loop(1, ROUNDS // 3, step, 0)
        visit(0, ROUNDS - 2, False, False)
        visit(1, ROUNDS - 1, False, True)
        plsc.subcore_barrier()

        for k in range(RPT // ZR):
            rs = pl.ds(row0 + k * ZR, ZR)
            pltpu.sync_copy(acc.at[rs], zbuf)

            @pl.when(cid == 0)
            def _():
                pltpu.sync_copy(zbuf, agg0.at[rs])

            @pl.when(cid == 1)
            def _():
                pltpu.sync_copy(zbuf, agg1.at[rs])

    return _scatter


@functools.cache
def _get_sc_counts():
    out_type = (
        jax.ShapeDtypeStruct((NPAD, D), jnp.float32),
        jax.ShapeDtypeStruct((NPAD, D), jnp.float32),
    )
    scratch = [
        pltpu.VMEM((3, CG), jnp.int32),
        pltpu.VMEM((CG, D), jnp.float32),
        pltpu.VMEM((ZR, D), jnp.float32),
        pltpu.VMEM_SHARED((NPAD, D), jnp.float32),
    ] + [pltpu.SemaphoreType.DMA] * 6
    mesh = plsc.VectorSubcoreMesh(core_axis_name="c", subcore_axis_name="s")

    @functools.partial(pl.kernel, out_type=out_type, mesh=mesh,
                       scratch_types=scratch)
    def _counts(dst2_hbm, cnt0, cnt1, idxr, ones, zbuf, acc,
                i0, i1, i2, a0, a1, a2):
        semix = (i0, i1, i2)
        semad = (a0, a1, a2)
        cid = lax.axis_index("c")
        sid = lax.axis_index("s")
        wid = sid * NC + cid
        row0 = sid * RPT

        def zrow(i, carry):
            for j in range(D // 16):
                zbuf[i, pl.ds(j * 16, 16)] = jnp.zeros((16,), jnp.float32)
            return carry

        lax.fori_loop(0, ZR, zrow, 0)

        def onesrow(i, carry):
            for j in range(D // 16):
                ones[i, pl.ds(j * 16, 16)] = jnp.ones((16,), jnp.float32)
            return carry

        lax.fori_loop(0, CG, onesrow, 0)
        for k in range(RPT // ZR):
            pltpu.sync_copy(zbuf, acc.at[pl.ds(row0 + k * ZR, ZR)])
        plsc.subcore_barrier()

        def fire_idx(b, r):
            return pltpu.async_copy(dst2_hbm.at[wid].at[pl.ds(r, 1)],
                                    idxr.at[pl.ds(b, 1)], semix[b])

        def wait_idx(b, r):
            pltpu.make_async_copy(dst2_hbm.at[wid].at[pl.ds(r, 1)],
                                  idxr.at[pl.ds(b, 1)], semix[b]).wait()

        def fire_add(b):
            pltpu.async_copy(ones, acc.at[idxr.at[b]], semad[b], add=True)

        def wait_add(b):
            pltpu.make_async_copy(ones, acc.at[idxr.at[b]], semad[b]).wait()

        fire_idx(0, 0)
        fire_idx(1, 1)

        def visit(s, r, first, last):
            wait_idx(s, r)
            fire_add(s)
            s1 = (s + 2) % 3
            if first:
                fire_idx(s1, r + 2)
            else:
                wait_add(s1)

                @pl.when(r + 2 < ROUNDS)
                def _():
                    fire_idx(s1, r + 2)
            if last:
                wait_add(s)

        def step(g, carry):
            for b in range(3):
                visit(b, g * 3 + b, False, False)
            return carry

        visit(0, 0, True, False)
        visit(1, 1, False, False)
        visit(2, 2, False, False)
        lax.fori_loop(1, ROUNDS // 3, step, 0)
        visit(0, ROUNDS - 2, False, False)
        visit(1, ROUNDS - 1, False, True)
        plsc.subcore_barrier()

        for k in range(RPT // ZR):
            rs = pl.ds(row0 + k * ZR, ZR)
            pltpu.sync_copy(acc.at[rs], zbuf)

            @pl.when(cid == 0)
            def _():
                pltpu.sync_copy(zbuf, cnt0.at[rs])

            @pl.when(cid == 1)
            def _():
                pltpu.sync_copy(zbuf, cnt1.at[rs])

    return _counts


def kernel(v, c_latent, e_latent, params, edge_index, batch):
    src = edge_index[0].astype(jnp.int32)
    dst = edge_index[1].astype(jnp.int32)
    p = params
    blk0, blk1 = p['block0'], p['block1']

    def b2(b):
        return b.reshape(1, D)

    We1_0, We1_1 = blk0['We1'], blk1['We1']
    Wn1_0, Wn1_1 = blk0['Wn1'], blk1['Wn1']
    Wo = jnp.pad(p['W_out'], ((0, 0), (0, D - p['W_out'].shape[1])))
    bo = jnp.pad(p['b_out'], (0, D - p['b_out'].shape[0])).reshape(1, D)

    h0, P0, Q0 = _node_init(v, c_latent, p['W_in'], b2(p['b_in']),
                            p['W_cond'], b2(p['b_cond']),
                            We1_0[D:2 * D], b2(blk0['be1']), We1_0[2 * D:])
    src2 = src.reshape(NW, ROUNDS, CG)
    dst2 = dst.reshape(NW, ROUNDS, CG)
    G0 = _get_sc_gather()(P0, Q0, src2, dst2)
    e1 = _edge_mlp(e_latent, G0, We1_0[:D], blk0['We2'], b2(blk0['be2']),
                   Wed=p['W_edge'], bed=b2(p['b_edge']))
    c0, c1 = _get_sc_counts()(dst2)
    a0, a1 = _get_sc_scatter()(e1, dst2)
    h1, P1, Q1 = _node_mid(h0, a0, a1, c0, c1, Wn1_0[:D], Wn1_0[D:],
                           b2(blk0['bn1']), blk0['Wn2'], b2(blk0['bn2']),
                           We1_1[D:2 * D], b2(blk1['be1']), We1_1[2 * D:])
    G1 = _get_sc_gather()(P1, Q1, src2, dst2)
    e2 = _edge_mlp(e1, G1, We1_1[:D], blk1['We2'], b2(blk1['be2']))
    a0b, a1b = _get_sc_scatter()(e2, dst2)
    out_pad = _node_final(h1, a0b, a1b, c0, c1, Wn1_1[:D], Wn1_1[D:],
                          b2(blk1['bn1']), blk1['Wn2'], b2(blk1['bn2']),
                          Wo, bo)
    return out_pad[:, :3]


# speedup vs baseline: 1.0447x; 1.0099x over previous
import functools

import jax
import jax.numpy as jnp
from jax import lax
from jax.experimental import pallas as pl
from jax.experimental.pallas import tpu as pltpu
from jax.experimental.pallas import tpu_sc as plsc

N = 10000
E = 320000
D = 128
NC = 2
NS = 16
NW = NC * NS
EPW = E // NW
CG = 80
ROUNDS = EPW // CG
NPAD = 10240
RPT = NPAD // NS
ZR = 16

_SELU_ALPHA = 1.6732632423543772
_SELU_SCALE = 1.0507009873554805


def _selu(x):
    return _SELU_SCALE * jnp.where(x > 0, x, _SELU_ALPHA * (jnp.exp(x) - 1.0))


def _bmm(x, w):
    return jnp.dot(x.astype(jnp.bfloat16), w.astype(jnp.bfloat16),
                   preferred_element_type=jnp.float32)


def _node_init_body(v_ref, c_ref, Wi_ref, bi_ref, Wc_ref, bc_ref,
                    Ws_ref, be1_ref, Wd_ref, h_ref, p_ref, q_ref):
    h = (jnp.dot(v_ref[...], Wi_ref[...], preferred_element_type=jnp.float32)
         + bi_ref[...]
         + jnp.dot(c_ref[...], Wc_ref[...], preferred_element_type=jnp.float32)
         + bc_ref[...])
    h_ref[...] = h
    p_ref[...] = jnp.dot(h, Ws_ref[...], preferred_element_type=jnp.float32) + be1_ref[...]
    q_ref[...] = jnp.dot(h, Wd_ref[...], preferred_element_type=jnp.float32)


def _node_init(v, c, Wi, bi, Wc, bc, Ws, be1, Wd):
    BN = 2000
    grid = (N // BN,)
    row = pl.BlockSpec((BN, D), lambda i: (i, 0))
    full = pl.BlockSpec((D, D), lambda i: (0, 0))
    bias = pl.BlockSpec((1, D), lambda i: (0, 0))
    return pl.pallas_call(
        _node_init_body,
        grid=grid,
        in_specs=[row, row, full, bias, full, bias, full, bias, full],
        out_specs=[row, row, row],
        out_shape=[jax.ShapeDtypeStruct((N, D), jnp.float32)] * 3,
    )(v, c, Wi, bi, Wc, bc, Ws, be1, Wd)


def _edge_body_first(el_ref, g_ref, Wed_ref, bed_ref, We_ref, We2_ref,
                     be2_ref, out_ref):
    e = _bmm(el_ref[...], Wed_ref[...]) + bed_ref[...]
    t = _selu(_bmm(e, We_ref[...]) + g_ref[...])
    out_ref[...] = _bmm(t, We2_ref[...]) + be2_ref[...] + e


def _edge_body_mid(e_ref, g_ref, We_ref, We2_ref, be2_ref, out_ref):
    e = e_ref[...]
    t = _selu(_bmm(e, We_ref[...]) + g_ref[...])
    out_ref[...] = _bmm(t, We2_ref[...]) + be2_ref[...] + e


def _edge_mlp(e_in, G, We, We2, be2, Wed=None, bed=None):
    BE = 2560
    grid = (E // BE,)
    row = pl.BlockSpec((BE, D), lambda i: (i, 0))
    full = pl.BlockSpec((D, D), lambda i: (0, 0))
    bias = pl.BlockSpec((1, D), lambda i: (0, 0))
    if Wed is not None:
        return pl.pallas_call(
            _edge_body_first,
            grid=grid,
            in_specs=[row, row, full, bias, full, full, bias],
            out_specs=row,
            out_shape=jax.ShapeDtypeStruct((E, D), jnp.float32),
        )(e_in, G, Wed, bed, We, We2, be2)
    return pl.pallas_call(
        _edge_body_mid,
        grid=grid,
        in_specs=[row, row, full, full, bias],
        out_specs=row,
        out_shape=jax.ShapeDtypeStruct((E, D), jnp.float32),
    )(e_in, G, We, We2, be2)


def _node_mid_body(h_ref, a0_ref, a1_ref, c0_ref, c1_ref, Wh_ref, Wa_ref,
                   bn1_ref, Wn2_ref, bn2_ref, Ws_ref, be1_ref, Wd_ref,
                   h_out, p_out, q_out):
    cnt = c0_ref[...][:, 0:1] + c1_ref[...][:, 0:1]
    agg = (a0_ref[...] + a1_ref[...]) / jnp.maximum(cnt, 1.0)
    h = h_ref[...]
    t = _selu(jnp.dot(h, Wh_ref[...], preferred_element_type=jnp.float32)
              + jnp.dot(agg, Wa_ref[...], preferred_element_type=jnp.float32)
              + bn1_ref[...])
    hn = jnp.dot(t, Wn2_ref[...], preferred_element_type=jnp.float32) + bn2_ref[...] + h
    h_out[...] = hn
    p_out[...] = jnp.dot(hn, Ws_ref[...], preferred_element_type=jnp.float32) + be1_ref[...]
    q_out[...] = jnp.dot(hn, Wd_ref[...], preferred_element_type=jnp.float32)


def _node_mid(h, a0, a1, c0, c1, Wh, Wa, bn1, Wn2, bn2, Ws, be1, Wd):
    BN = 2000
    grid = (N // BN,)
    row = pl.BlockSpec((BN, D), lambda i: (i, 0))
    crow = pl.BlockSpec((BN, D), lambda i: (i, 0))
    full = pl.BlockSpec((D, D), lambda i: (0, 0))
    bias = pl.BlockSpec((1, D), lambda i: (0, 0))
    return pl.pallas_call(
        _node_mid_body,
        grid=grid,
        in_specs=[row, row, row, crow, crow, full, full, bias, full, bias,
                  full, bias, full],
        out_specs=[row, row, row],
        out_shape=[jax.ShapeDtypeStruct((N, D), jnp.float32)] * 3,
    )(h, a0, a1, c0, c1, Wh, Wa, bn1, Wn2, bn2, Ws, be1, Wd)


def _node_final_body(h_ref, a0_ref, a1_ref, c0_ref, c1_ref, Wh_ref, Wa_ref,
                     bn1_ref, Wn2_ref, bn2_ref, Wo_ref, bo_ref, out_ref):
    cnt = c0_ref[...][:, 0:1] + c1_ref[...][:, 0:1]
    agg = (a0_ref[...] + a1_ref[...]) / jnp.maximum(cnt, 1.0)
    h = h_ref[...]
    t = _selu(jnp.dot(h, Wh_ref[...], preferred_element_type=jnp.float32)
              + jnp.dot(agg, Wa_ref[...], preferred_element_type=jnp.float32)
              + bn1_ref[...])
    hn = jnp.dot(t, Wn2_ref[...], preferred_element_type=jnp.float32) + bn2_ref[...] + h
    out_ref[...] = jnp.dot(hn, Wo_ref[...], preferred_element_type=jnp.float32) + bo_ref[...]


def _node_final(h, a0, a1, c0, c1, Wh, Wa, bn1, Wn2, bn2, Wo, bo):
    BN = 2000
    grid = (N // BN,)
    row = pl.BlockSpec((BN, D), lambda i: (i, 0))
    crow = pl.BlockSpec((BN, D), lambda i: (i, 0))
    full = pl.BlockSpec((D, D), lambda i: (0, 0))
    bias = pl.BlockSpec((1, D), lambda i: (0, 0))
    return pl.pallas_call(
        _node_final_body,
        grid=grid,
        in_specs=[row, row, row, crow, crow, full, full, bias, full, bias,
                  full, bias],
        out_specs=row,
        out_shape=jax.ShapeDtypeStruct((N, D), jnp.float32),
    )(h, a0, a1, c0, c1, Wh, Wa, bn1, Wn2, bn2, Wo, bo)


@functools.cache
def _get_sc_gather():
    mesh = plsc.VectorSubcoreMesh(core_axis_name="c", subcore_axis_name="s")

    @functools.partial(
        pl.kernel,
        out_type=jax.ShapeDtypeStruct((E, D), jnp.float32),
        mesh=mesh,
        scratch_types=[
            pltpu.VMEM((ROUNDS, CG), jnp.int32),
            pltpu.VMEM((ROUNDS, CG), jnp.int32),
            pltpu.VMEM((3, CG, D), jnp.float32),
            pltpu.VMEM((3, CG, D), jnp.float32),
            pltpu.VMEM((3, CG, D), jnp.float32),
            pltpu.SemaphoreType.DMA,
            pltpu.SemaphoreType.DMA,
            pltpu.SemaphoreType.DMA,
            pltpu.SemaphoreType.DMA,
            pltpu.SemaphoreType.DMA,
            pltpu.SemaphoreType.DMA,
            pltpu.SemaphoreType.DMA,
            pltpu.SemaphoreType.DMA,
            pltpu.SemaphoreType.DMA,
        ],
    )
    def _sc_gather(p_hbm, q_hbm, src2_hbm, dst2_hbm, g_hbm, idxs, idxd,
                   bufp, bufq, bufs, semp0, semp1, semp2, semq0, semq1,
                   semq2, sems0, sems1, sems2):
        wid = lax.axis_index("s") * NC + lax.axis_index("c")
        base0 = wid * EPW
        semp = (semp0, semp1, semp2)
        semq = (semq0, semq1, semq2)
        sems = (sems0, sems1, sems2)

        pltpu.sync_copy(src2_hbm.at[wid], idxs)
        pltpu.sync_copy(dst2_hbm.at[wid], idxd)

        def fire(b, r):
            pltpu.async_copy(p_hbm.at[idxs.at[r]], bufp.at[b], semp[b])
            pltpu.async_copy(q_hbm.at[idxd.at[r]], bufq.at[b], semq[b])

        def wait_gather(b, r):
            pltpu.make_async_copy(p_hbm.at[idxs.at[r]], bufp.at[b], semp[b]).wait()
            pltpu.make_async_copy(q_hbm.at[idxd.at[r]], bufq.at[b], semq[b]).wait()

        def wait_store(b, r):
            base = pl.multiple_of(base0 + r * CG, CG)
            pltpu.make_async_copy(bufs.at[b], g_hbm.at[pl.ds(base, CG)],
                                  sems[b]).wait()

        def fire_store(b, r):
            base = pl.multiple_of(base0 + r * CG, CG)
            return pltpu.async_copy(bufs.at[b], g_hbm.at[pl.ds(base, CG)],
                                    sems[b])

        def add(b):
            def add_row(i, carry):
                for j in range(D // 16):
                    sl = pl.ds(j * 16, 16)
                    bufs[b, i, sl] = bufp[b, i, sl] + bufq[b, i, sl]
                return carry

            lax.fori_loop(0, CG, add_row, 0)

        fire(0, 0)
        fire(1, 1)
        fire(2, 2)

        def visit(s, r):
            wait_gather(s, r)

            @pl.when(r >= 3)
            def _():
                wait_store(s, r - 3)

            add(s)

            @pl.when(r + 3 < ROUNDS)
            def _():
                fire(s, r + 3)

            fire_store(s, r)

        def step(g, carry):
            for b in range(3):
                visit(b, g * 3 + b)
            return carry

        lax.fori_loop(0, ROUNDS // 3, step, 0)
        visit(0, ROUNDS - 2)
        visit(1, ROUNDS - 1)
        wait_store(2, ROUNDS - 3)
        wait_store(0, ROUNDS - 2)
        wait_store(1, ROUNDS - 1)

    return _sc_gather


@functools.cache
def _get_sc_scatter():
    out_type = (
        jax.ShapeDtypeStruct((NPAD, D), jnp.float32),
        jax.ShapeDtypeStruct((NPAD, D), jnp.float32),
    )
    scratch = [
        pltpu.VMEM((3, CG), jnp.int32),
        pltpu.VMEM((3, CG, D), jnp.float32),
        pltpu.VMEM((ZR, D), jnp.float32),
        pltpu.VMEM_SHARED((NPAD, D), jnp.float32),
    ] + [pltpu.SemaphoreType.DMA] * 9
    mesh = plsc.VectorSubcoreMesh(core_axis_name="c", subcore_axis_name="s")

    @functools.partial(pl.kernel, out_type=out_type, mesh=mesh,
                       scratch_types=scratch)
    def _scatter(e_hbm, dst2_hbm, agg0, agg1, idxr, ebuf, zbuf, acc,
                 l0, l1, l2, i0, i1, i2, s0, s1, s2):
        semld = (l0, l1, l2)
        semix = (i0, i1, i2)
        semsc = (s0, s1, s2)
        cid = lax.axis_index("c")
        sid = lax.axis_index("s")
        wid = sid * NC + cid
        base0 = wid * EPW
        row0 = sid * RPT

        def zrow(i, carry):
            for j in range(D // 16):
                zbuf[i, pl.ds(j * 16, 16)] = jnp.zeros((16,), jnp.float32)
            return carry

        lax.fori_loop(0, ZR, zrow, 0)
        for k in range(RPT // ZR):
            pltpu.sync_copy(zbuf, acc.at[pl.ds(row0 + k * ZR, ZR)])

        plsc.subcore_barrier()

        def fire_load(b, r):
            base = pl.multiple_of(base0 + r * CG, CG)
            pltpu.async_copy(dst2_hbm.at[wid].at[pl.ds(r, 1)],
                             idxr.at[pl.ds(b, 1)], semix[b])
            return pltpu.async_copy(e_hbm.at[pl.ds(base, CG)], ebuf.at[b],
                                    semld[b])

        def wait_load(b, r):
            base = pl.multiple_of(base0 + r * CG, CG)
            pltpu.make_async_copy(e_hbm.at[pl.ds(base, CG)], ebuf.at[b],
                                  semld[b]).wait()
            pltpu.make_async_copy(dst2_hbm.at[wid].at[pl.ds(r, 1)],
                                  idxr.at[pl.ds(b, 1)], semix[b]).wait()

        def fire_scatter(b):
            pltpu.async_copy(ebuf.at[b], acc.at[idxr.at[b]], semsc[b],
                             add=True)

        def wait_scatter(b):
            pltpu.make_async_copy(ebuf.at[b], acc.at[idxr.at[b]],
                                  semsc[b]).wait()

        fire_load(0, 0)
        fire_load(1, 1)

        def visit(s, r, first, last):
            wait_load(s, r)
            fire_scatter(s)
            s1 = (s + 2) % 3
            if first:
                fire_load(s1, r + 2)
            else:
                wait_scatter(s1)

                @pl.when(r + 2 < ROUNDS)
                def _():
                    fire_load(s1, r + 2)
            if last:
                wait_scatter(s)

        def step(g, carry):
            for b in range(3):
                visit(b, g * 3 + b, False, False)
            return carry

        visit(0, 0, True, False)
        visit(1, 1, False, False)
        visit(2, 2, False, False)
        lax.fori_loop(1, ROUNDS // 3, step, 0)
        visit(0, ROUNDS - 2, False, False)
        visit(1, ROUNDS - 1, False, True)
        plsc.subcore_barrier()

        for k in range(RPT // ZR):
            rs = pl.ds(row0 + k * ZR, ZR)
            pltpu.sync_copy(acc.at[rs], zbuf)

            @pl.when(cid == 0)
            def _():
                pltpu.sync_copy(zbuf, agg0.at[rs])

            @pl.when(cid == 1)
            def _():
                pltpu.sync_copy(zbuf, agg1.at[rs])

    return _scatter


@functools.cache
def _get_sc_counts():
    out_type = (
        jax.ShapeDtypeStruct((NPAD, D), jnp.float32),
        jax.ShapeDtypeStruct((NPAD, D), jnp.float32),
    )
    scratch = [
        pltpu.VMEM((3, CG), jnp.int32),
        pltpu.VMEM((CG, D), jnp.float32),
        pltpu.VMEM((ZR, D), jnp.float32),
        pltpu.VMEM_SHARED((NPAD, D), jnp.float32),
    ] + [pltpu.SemaphoreType.DMA] * 6
    mesh = plsc.VectorSubcoreMesh(core_axis_name="c", subcore_axis_name="s")

    @functools.partial(pl.kernel, out_type=out_type, mesh=mesh,
                       scratch_types=scratch)
    def _counts(dst2_hbm, cnt0, cnt1, idxr, ones, zbuf, acc,
                i0, i1, i2, a0, a1, a2):
        semix = (i0, i1, i2)
        semad = (a0, a1, a2)
        cid = lax.axis_index("c")
        sid = lax.axis_index("s")
        wid = sid * NC + cid
        row0 = sid * RPT

        def zrow(i, carry):
            for j in range(D // 16):
                zbuf[i, pl.ds(j * 16, 16)] = jnp.zeros((16,), jnp.float32)
            return carry

        lax.fori_loop(0, ZR, zrow, 0)

        def onesrow(i, carry):
            for j in range(D // 16):
                ones[i, pl.ds(j * 16, 16)] = jnp.ones((16,), jnp.float32)
            return carry

        lax.fori_loop(0, CG, onesrow, 0)
        for k in range(RPT // ZR):
            pltpu.sync_copy(zbuf, acc.at[pl.ds(row0 + k * ZR, ZR)])
        plsc.subcore_barrier()

        def fire_idx(b, r):
            return pltpu.async_copy(dst2_hbm.at[wid].at[pl.ds(r, 1)],
                                    idxr.at[pl.ds(b, 1)], semix[b])

        def wait_idx(b, r):
            pltpu.make_async_copy(dst2_hbm.at[wid].at[pl.ds(r, 1)],
                                  idxr.at[pl.ds(b, 1)], semix[b]).wait()

        def fire_add(b):
            pltpu.async_copy(ones, acc.at[idxr.at[b]], semad[b], add=True)

        def wait_add(b):
            pltpu.make_async_copy(ones, acc.at[idxr.at[b]], semad[b]).wait()

        fire_idx(0, 0)
        fire_idx(1, 1)

        def visit(s, r, first, last):
            wait_idx(s, r)
            fire_add(s)
            s1 = (s + 2) % 3
            if first:
                fire_idx(s1, r + 2)
            else:
                wait_add(s1)

                @pl.when(r + 2 < ROUNDS)
                def _():
                    fire_idx(s1, r + 2)
            if last:
                wait_add(s)

        def step(g, carry):
            for b in range(3):
                visit(b, g * 3 + b, False, False)
            return carry

        visit(0, 0, True, False)
        visit(1, 1, False, False)
        visit(2, 2, False, False)
        lax.fori_loop(1, ROUNDS // 3, step, 0)
        visit(0, ROUNDS - 2, False, False)
        visit(1, ROUNDS - 1, False, True)
        plsc.subcore_barrier()

        for k in range(RPT // ZR):
            rs = pl.ds(row0 + k * ZR, ZR)
            pltpu.sync_copy(acc.at[rs], zbuf)

            @pl.when(cid == 0)
            def _():
                pltpu.sync_copy(zbuf, cnt0.at[rs])

            @pl.when(cid == 1)
            def _():
                pltpu.sync_copy(zbuf, cnt1.at[rs])

    return _counts


def kernel(v, c_latent, e_latent, params, edge_index, batch):
    src = edge_index[0].astype(jnp.int32)
    dst = edge_index[1].astype(jnp.int32)
    p = params
    blk0, blk1 = p['block0'], p['block1']

    def b2(b):
        return b.reshape(1, D)

    We1_0, We1_1 = blk0['We1'], blk1['We1']
    Wn1_0, Wn1_1 = blk0['Wn1'], blk1['Wn1']
    Wo = jnp.pad(p['W_out'], ((0, 0), (0, D - p['W_out'].shape[1])))
    bo = jnp.pad(p['b_out'], (0, D - p['b_out'].shape[0])).reshape(1, D)

    h0, P0, Q0 = _node_init(v, c_latent, p['W_in'], b2(p['b_in']),
                            p['W_cond'], b2(p['b_cond']),
                            We1_0[D:2 * D], b2(blk0['be1']), We1_0[2 * D:])
    src2 = src.reshape(NW, ROUNDS, CG)
    dst2 = dst.reshape(NW, ROUNDS, CG)
    G0 = _get_sc_gather()(P0, Q0, src2, dst2)
    e1 = _edge_mlp(e_latent, G0, We1_0[:D], blk0['We2'], b2(blk0['be2']),
                   Wed=p['W_edge'], bed=b2(p['b_edge']))
    c0, c1 = _get_sc_counts()(dst2)
    a0, a1 = _get_sc_scatter()(e1, dst2)
    h1, P1, Q1 = _node_mid(h0, a0, a1, c0, c1, Wn1_0[:D], Wn1_0[D:],
                           b2(blk0['bn1']), blk0['Wn2'], b2(blk0['bn2']),
                           We1_1[D:2 * D], b2(blk1['be1']), We1_1[2 * D:])
    G1 = _get_sc_gather()(P1, Q1, src2, dst2)
    e2 = _edge_mlp(e1, G1, We1_1[:D], blk1['We2'], b2(blk1['be2']))
    a0b, a1b = _get_sc_scatter()(e2, dst2)
    out_pad = _node_final(h1, a0b, a1b, c0, c1, Wn1_1[:D], Wn1_1[D:],
                          b2(blk1['bn1']), blk1['Wn2'], b2(blk1['bn2']),
                          Wo, bo)
    return out_pad[:, :3]
